# baseline (speedup 1.0000x reference)
import functools

import jax
import jax.numpy as jnp
import numpy as np
from jax import lax
from jax.experimental import pallas as pl
from jax.experimental.pallas import tpu as pltpu
from jax.experimental.pallas import tpu_sc as plsc

D = 2048
SEQ_IMG = 729
N_TOK = 4097
SEQ_OUT = N_TOK + SEQ_IMG

NW = 32
SUB = 24
NCHUNK = 6
CHUNK = SUB * NCHUNK
TOK_PAD = NW * CHUNK
IMG_PAD = NW * SUB

_TOK_DST = np.empty((TOK_PAD,), np.int32)
_TOK_DST[:2049] = np.arange(2049)
_TOK_DST[2049:N_TOK] = np.arange(2778, 4826)
_TOK_DST[N_TOK:] = _TOK_DST[1:1 + TOK_PAD - N_TOK]
_IMG_SRC = np.empty((IMG_PAD,), np.int32)
_IMG_SRC[:SEQ_IMG] = np.arange(SEQ_IMG)
_IMG_SRC[SEQ_IMG:] = _IMG_SRC[:IMG_PAD - SEQ_IMG]
_IMG_DST = np.empty((IMG_PAD,), np.int32)
_IMG_DST[:SEQ_IMG] = np.arange(2049, 2778)
_IMG_DST[SEQ_IMG:] = _IMG_DST[:IMG_PAD - SEQ_IMG]


def _sc_body(table_hbm, img3_hbm, tok_src_hbm, tok_dst_hbm, img_src_hbm,
             img_dst_hbm, out3_hbm, sidx_v, didx_v, iidx_v, idst_v, buf0, buf1,
             sem_in0, sem_in1, sem_out0, sem_out1):
    c = lax.axis_index("c")
    s = lax.axis_index("s")
    w = c * 16 + s
    img_hbm = img3_hbm.at[0]
    out_hbm = out3_hbm.at[0]

    pltpu.sync_copy(tok_src_hbm.at[w], sidx_v)
    pltpu.sync_copy(tok_dst_hbm.at[w], didx_v)
    pltpu.sync_copy(img_src_hbm.at[w], iidx_v)
    pltpu.sync_copy(img_dst_hbm.at[w], idst_v)

    chunks = [(table_hbm, sidx_v.at[j], didx_v.at[j]) for j in range(NCHUNK)]
    chunks.append((img_hbm, iidx_v.at[0], idst_v.at[0]))
    n = len(chunks)
    bufs = (buf0, buf1)
    sin = (sem_in0, sem_in1)
    sout = (sem_out0, sem_out1)

    gat = [None] * n
    sca = [None] * n
    src0, sidx0, _ = chunks[0]
    gat[0] = pltpu.async_copy(src0.at[sidx0], bufs[0], sin[0])
    for j in range(n):
        p = j % 2
        gat[j].wait()
        sca[j] = pltpu.async_copy(bufs[p], out_hbm.at[chunks[j][2]], sout[p])
        if j + 1 < n:
            if j >= 1:
                sca[j - 1].wait()
            src, sidx, _ = chunks[j + 1]
            gat[j + 1] = pltpu.async_copy(src.at[sidx], bufs[(j + 1) % 2],
                                          sin[(j + 1) % 2])
    sca[n - 2].wait()
    sca[n - 1].wait()


@functools.partial(
    pl.kernel,
    mesh=plsc.VectorSubcoreMesh(core_axis_name="c", subcore_axis_name="s"),
    out_type=jax.ShapeDtypeStruct((1, SEQ_OUT, D), jnp.float32),
    scratch_types=[
        pltpu.VMEM((NCHUNK, SUB), jnp.int32),
        pltpu.VMEM((NCHUNK, SUB), jnp.int32),
        pltpu.VMEM((1, SUB), jnp.int32),
        pltpu.VMEM((1, SUB), jnp.int32),
        pltpu.VMEM((SUB, D), jnp.float32),
        pltpu.VMEM((SUB, D), jnp.float32),
        pltpu.SemaphoreType.DMA,
        pltpu.SemaphoreType.DMA,
        pltpu.SemaphoreType.DMA,
        pltpu.SemaphoreType.DMA,
    ],
)
def _sc_gather(*refs):
    _sc_body(*refs)


def kernel(embed_table, image_embeds, before_ids, after_ids, bos_id):
    bos = jnp.asarray(bos_id, jnp.int32)
    tok_src = jnp.concatenate([
        bos[None],
        before_ids[0].astype(jnp.int32),
        after_ids[0].astype(jnp.int32),
    ])
    tok_src = jnp.concatenate([tok_src, tok_src[1:1 + TOK_PAD - N_TOK]])
    return _sc_gather(
        embed_table,
        image_embeds,
        tok_src.reshape(NW, NCHUNK, SUB),
        jnp.asarray(_TOK_DST.reshape(NW, NCHUNK, SUB)),
        jnp.asarray(_IMG_SRC.reshape(NW, 1, SUB)),
        jnp.asarray(_IMG_DST.reshape(NW, 1, SUB)),
    )


# speedup vs baseline: 1.0386x; 1.0386x over previous
import functools

import jax
import jax.numpy as jnp
import numpy as np
from jax import lax
from jax.experimental import pallas as pl
from jax.experimental.pallas import tpu as pltpu
from jax.experimental.pallas import tpu_sc as plsc

D = 2048
SEQ_IMG = 729
N_TOK = 4097
SEQ_OUT = N_TOK + SEQ_IMG

NW = 32
SUB = 24
NCHUNK = 6
CHUNK = SUB * NCHUNK
TOK_PAD = NW * CHUNK
IMG_PAD = NW * SUB

_TOK_DST = np.empty((TOK_PAD,), np.int32)
_TOK_DST[:2049] = np.arange(2049)
_TOK_DST[2049:N_TOK] = np.arange(2778, 4826)
_TOK_DST[N_TOK:] = _TOK_DST[1:1 + TOK_PAD - N_TOK]
_IMG_SRC = np.empty((IMG_PAD,), np.int32)
_IMG_SRC[:SEQ_IMG] = np.arange(SEQ_IMG)
_IMG_SRC[SEQ_IMG:] = _IMG_SRC[:IMG_PAD - SEQ_IMG]
_IMG_DST = np.empty((IMG_PAD,), np.int32)
_IMG_DST[:SEQ_IMG] = np.arange(2049, 2778)
_IMG_DST[SEQ_IMG:] = _IMG_DST[:IMG_PAD - SEQ_IMG]


def _sc_body(table_hbm, img3_hbm, tok_src_hbm, tok_dst_hbm, img_src_hbm,
             img_dst_hbm, out3_hbm, sidx_v, didx_v, iidx_v, idst_v, buf0, buf1,
             sem_in0, sem_in1, sem_out0, sem_out1):
    c = lax.axis_index("c")
    s = lax.axis_index("s")
    w = c * 16 + s
    img_hbm = img3_hbm
    out_hbm = out3_hbm.at[0]

    pltpu.sync_copy(tok_src_hbm.at[w], sidx_v)
    pltpu.sync_copy(tok_dst_hbm.at[w], didx_v)
    pltpu.sync_copy(img_src_hbm.at[w], iidx_v)
    pltpu.sync_copy(img_dst_hbm.at[w], idst_v)

    chunks = [(table_hbm, sidx_v.at[j], didx_v.at[j]) for j in range(NCHUNK)]
    chunks.append((img_hbm, iidx_v.at[0], idst_v.at[0]))
    n = len(chunks)
    bufs = (buf0, buf1)
    sin = (sem_in0, sem_in1)
    sout = (sem_out0, sem_out1)

    gat = [None] * n
    sca = [None] * n
    src0, sidx0, _ = chunks[0]
    gat[0] = pltpu.async_copy(src0.at[sidx0], bufs[0], sin[0])
    for j in range(n):
        p = j % 2
        gat[j].wait()
        sca[j] = pltpu.async_copy(bufs[p], out_hbm.at[chunks[j][2]], sout[p])
        if j + 1 < n:
            if j >= 1:
                sca[j - 1].wait()
            src, sidx, _ = chunks[j + 1]
            gat[j + 1] = pltpu.async_copy(src.at[sidx], bufs[(j + 1) % 2],
                                          sin[(j + 1) % 2])
    sca[n - 2].wait()
    sca[n - 1].wait()


@functools.partial(
    pl.kernel,
    mesh=plsc.VectorSubcoreMesh(core_axis_name="c", subcore_axis_name="s"),
    out_type=jax.ShapeDtypeStruct((1, SEQ_OUT, D), jnp.float32),
    scratch_types=[
        pltpu.VMEM((NCHUNK, SUB), jnp.int32),
        pltpu.VMEM((NCHUNK, SUB), jnp.int32),
        pltpu.VMEM((1, SUB), jnp.int32),
        pltpu.VMEM((1, SUB), jnp.int32),
        pltpu.VMEM((SUB, D), jnp.float32),
        pltpu.VMEM((SUB, D), jnp.float32),
        pltpu.SemaphoreType.DMA,
        pltpu.SemaphoreType.DMA,
        pltpu.SemaphoreType.DMA,
        pltpu.SemaphoreType.DMA,
    ],
)
def _sc_gather(*refs):
    _sc_body(*refs)


def kernel(embed_table, image_embeds, before_ids, after_ids, bos_id):
    bos = jnp.asarray(bos_id, jnp.int32)
    tok_src = jnp.concatenate([
        bos[None],
        before_ids[0].astype(jnp.int32),
        after_ids[0].astype(jnp.int32),
    ])
    tok_src = jnp.concatenate([tok_src, tok_src[1:1 + TOK_PAD - N_TOK]])
    return _sc_gather(
        embed_table,
        image_embeds[0],
        tok_src.reshape(NW, NCHUNK, SUB),
        jnp.asarray(_TOK_DST.reshape(NW, NCHUNK, SUB)),
        jnp.asarray(_IMG_SRC.reshape(NW, 1, SUB)),
        jnp.asarray(_IMG_DST.reshape(NW, 1, SUB)),
    )


# speedup vs baseline: 1.4462x; 1.3925x over previous
import functools

import jax
import jax.numpy as jnp
import numpy as np
from jax import lax
from jax.experimental import pallas as pl
from jax.experimental.pallas import tpu as pltpu
from jax.experimental.pallas import tpu_sc as plsc

D = 2048
SEQ_IMG = 729
N_TOK = 4097
SEQ_OUT = N_TOK + SEQ_IMG

NW = 32
SUB = 24
NCHUNK = 6
CHUNK = SUB * NCHUNK
TOK_PAD = NW * CHUNK
IMG_PAD = NW * SUB

_TOK_DST = np.empty((TOK_PAD,), np.int32)
_TOK_DST[:2049] = np.arange(2049)
_TOK_DST[2049:N_TOK] = np.arange(2778, 4826)
_TOK_DST[N_TOK:] = _TOK_DST[1:1 + TOK_PAD - N_TOK]
_IMG_SRC = np.empty((IMG_PAD,), np.int32)
_IMG_SRC[:SEQ_IMG] = np.arange(SEQ_IMG)
_IMG_SRC[SEQ_IMG:] = _IMG_SRC[:IMG_PAD - SEQ_IMG]
_IMG_DST = np.empty((IMG_PAD,), np.int32)
_IMG_DST[:SEQ_IMG] = np.arange(2049, 2778)
_IMG_DST[SEQ_IMG:] = _IMG_DST[:IMG_PAD - SEQ_IMG]


def _sc_body(table_hbm, img3_hbm, tok_src_hbm, tok_dst_hbm, img_src_hbm,
             img_dst_hbm, out3_hbm, sidx_v, didx_v, iidx_v, idst_v, buf0, buf1,
             sem_in0, sem_in1, sem_out0, sem_out1):
    c = lax.axis_index("c")
    s = lax.axis_index("s")
    w = c * 16 + s
    img_hbm = img3_hbm
    out_hbm = out3_hbm.at[0]

    pltpu.sync_copy(tok_src_hbm.at[w], sidx_v)
    pltpu.sync_copy(tok_dst_hbm.at[w], didx_v)
    pltpu.sync_copy(img_src_hbm.at[w], iidx_v)
    pltpu.sync_copy(img_dst_hbm.at[w], idst_v)

    chunks = [(table_hbm, sidx_v.at[j], didx_v.at[j]) for j in range(NCHUNK)]
    chunks.append((img_hbm, iidx_v.at[0], idst_v.at[0]))
    n = len(chunks)
    bufs = (buf0, buf1)
    sin = (sem_in0, sem_in1)
    sout = (sem_out0, sem_out1)

    gat = [None] * n
    sca = [None] * n
    src0, sidx0, _ = chunks[0]
    gat[0] = pltpu.async_copy(src0.at[sidx0], bufs[0], sin[0])
    for j in range(n):
        p = j % 2
        gat[j].wait()
        sca[j] = pltpu.async_copy(bufs[p], out_hbm.at[chunks[j][2]], sout[p])
        if j + 1 < n:
            if j >= 1:
                sca[j - 1].wait()
            src, sidx, _ = chunks[j + 1]
            gat[j + 1] = pltpu.async_copy(src.at[sidx], bufs[(j + 1) % 2],
                                          sin[(j + 1) % 2])
    sca[n - 2].wait()
    sca[n - 1].wait()


@functools.partial(
    pl.kernel,
    mesh=plsc.VectorSubcoreMesh(core_axis_name="c", subcore_axis_name="s"),
    out_type=jax.ShapeDtypeStruct((1, SEQ_OUT, D), jnp.float32),
    scratch_types=[
        pltpu.VMEM((NCHUNK, SUB), jnp.int32),
        pltpu.VMEM((NCHUNK, SUB), jnp.int32),
        pltpu.VMEM((1, SUB), jnp.int32),
        pltpu.VMEM((1, SUB), jnp.int32),
        pltpu.VMEM((SUB, D), jnp.float32),
        pltpu.VMEM((SUB, D), jnp.float32),
        pltpu.SemaphoreType.DMA,
        pltpu.SemaphoreType.DMA,
        pltpu.SemaphoreType.DMA,
        pltpu.SemaphoreType.DMA,
    ],
)
def _sc_gather(*refs):
    _sc_body(*refs)


def kernel(embed_table, image_embeds, before_ids, after_ids, bos_id):
    bos = jnp.asarray(bos_id, jnp.int32)
    tok_src = jnp.concatenate([
        bos[None],
        before_ids[0].astype(jnp.int32),
        after_ids[0].astype(jnp.int32),
    ])
    tok_src = jnp.concatenate([tok_src, tok_src[1:1 + TOK_PAD - N_TOK]])
    out = _sc_gather(
        embed_table,
        image_embeds[0],
        tok_src.reshape(NW, NCHUNK, SUB),
        jnp.asarray(_TOK_DST.reshape(NW, NCHUNK, SUB)),
        jnp.asarray(_IMG_SRC.reshape(NW, 1, SUB)),
        jnp.asarray(_IMG_DST.reshape(NW, 1, SUB)),
    )
    return lax.optimization_barrier(out)


# speedup vs baseline: 1.4478x; 1.0011x over previous
import functools

import jax
import jax.numpy as jnp
import numpy as np
from jax import lax
from jax.experimental import pallas as pl
from jax.experimental.pallas import tpu as pltpu
from jax.experimental.pallas import tpu_sc as plsc

D = 2048
SEQ_IMG = 729
N_TOK = 4097
SEQ_OUT = N_TOK + SEQ_IMG

NW = 32
SUB = 16
NCHUNK = 8
NBUF = 3
IMG_CHUNKS = 46
IMG_SECOND = IMG_CHUNKS - NW

_DST = np.empty((N_TOK,), np.int32)
_DST[:2049] = np.arange(2049)
_DST[2049:] = np.arange(2778, 4826)

_TOK_DST3 = np.empty((NW, NCHUNK + 1, SUB), np.int32)
_TOK_DST3[:, :NCHUNK, :] = _DST[:4096].reshape(NW, NCHUNK, SUB)
_TOK_DST3[:, NCHUNK, :] = _DST[4096]

_IMG_SRC_PAD = np.concatenate([np.arange(SEQ_IMG),
                               np.arange(SEQ_IMG - 7, SEQ_IMG)]).astype(np.int32)
_IMG_DST_PAD = _IMG_SRC_PAD + 2049
_IMG_SRC3 = np.empty((NW, 2, SUB), np.int32)
_IMG_DST3 = np.empty((NW, 2, SUB), np.int32)
_IMG_SRC3[:, 0, :] = _IMG_SRC_PAD[:NW * SUB].reshape(NW, SUB)
_IMG_DST3[:, 0, :] = _IMG_DST_PAD[:NW * SUB].reshape(NW, SUB)
_IMG_SRC3[:, 1, :] = 0
_IMG_DST3[:, 1, :] = 2049
_IMG_SRC3[:IMG_SECOND, 1, :] = _IMG_SRC_PAD[NW * SUB:].reshape(IMG_SECOND, SUB)
_IMG_DST3[:IMG_SECOND, 1, :] = _IMG_DST_PAD[NW * SUB:].reshape(IMG_SECOND, SUB)


def _sc_body(table_hbm, img_hbm, tok_src_hbm, tok_dst_hbm, img_src_hbm,
             img_dst_hbm, out_hbm, sidx_v, didx_v, iidx_v, idst_v,
             buf0, buf1, buf2, si0, si1, si2, so0, so1, so2):
    c = lax.axis_index("c")
    s = lax.axis_index("s")
    w = c * 16 + s

    pltpu.sync_copy(tok_src_hbm.at[w], sidx_v)
    pltpu.sync_copy(tok_dst_hbm.at[w], didx_v)
    pltpu.sync_copy(img_src_hbm.at[w], iidx_v)
    pltpu.sync_copy(img_dst_hbm.at[w], idst_v)

    chunks = [(table_hbm, sidx_v.at[j], didx_v.at[j]) for j in range(NCHUNK)]
    chunks.append((img_hbm, iidx_v.at[0], idst_v.at[0]))
    n = len(chunks)
    bufs = (buf0, buf1, buf2)
    sin = (si0, si1, si2)
    sout = (so0, so1, so2)

    gat = [None] * n
    sca = [None] * n
    for k in range(min(NBUF, n)):
        src, sidx, _ = chunks[k]
        gat[k] = pltpu.async_copy(src.at[sidx], bufs[k % NBUF], sin[k % NBUF])
    for j in range(n):
        p = j % NBUF
        gat[j].wait()
        sca[j] = pltpu.async_copy(bufs[p], out_hbm.at[chunks[j][2]], sout[p])
        k = j + NBUF
        if k < n:
            sca[j].wait()
            src, sidx, _ = chunks[k]
            gat[k] = pltpu.async_copy(src.at[sidx], bufs[p], sin[p])
    for j in range(max(0, n - NBUF + 1), n):
        sca[j].wait()

    @pl.when(w == 0)
    def _():
        pltpu.async_copy(table_hbm.at[sidx_v.at[NCHUNK]], buf0, si0).wait()
        pltpu.async_copy(buf0, out_hbm.at[didx_v.at[NCHUNK]], so0).wait()

    @pl.when(w < IMG_SECOND)
    def _():
        pltpu.async_copy(img_hbm.at[iidx_v.at[1]], buf1, si1).wait()
        pltpu.async_copy(buf1, out_hbm.at[idst_v.at[1]], so1).wait()


@functools.partial(
    pl.kernel,
    mesh=plsc.VectorSubcoreMesh(core_axis_name="c", subcore_axis_name="s"),
    out_type=jax.ShapeDtypeStruct((1, SEQ_OUT, D), jnp.float32),
    scratch_types=[
        pltpu.VMEM((NCHUNK + 1, SUB), jnp.int32),
        pltpu.VMEM((NCHUNK + 1, SUB), jnp.int32),
        pltpu.VMEM((2, SUB), jnp.int32),
        pltpu.VMEM((2, SUB), jnp.int32),
        pltpu.VMEM((SUB, D), jnp.float32),
        pltpu.VMEM((SUB, D), jnp.float32),
        pltpu.VMEM((SUB, D), jnp.float32),
        pltpu.SemaphoreType.DMA,
        pltpu.SemaphoreType.DMA,
        pltpu.SemaphoreType.DMA,
        pltpu.SemaphoreType.DMA,
        pltpu.SemaphoreType.DMA,
        pltpu.SemaphoreType.DMA,
    ],
)
def _sc_gather(table_hbm, img_hbm, tok_src_hbm, tok_dst_hbm, img_src_hbm,
               img_dst_hbm, out3_hbm, *rest):
    _sc_body(table_hbm, img_hbm, tok_src_hbm, tok_dst_hbm, img_src_hbm,
             img_dst_hbm, out3_hbm.at[0], *rest)


def kernel(embed_table, image_embeds, before_ids, after_ids, bos_id):
    bos = jnp.asarray(bos_id, jnp.int32)
    tok_src = jnp.concatenate([
        bos[None],
        before_ids[0].astype(jnp.int32),
        after_ids[0].astype(jnp.int32),
    ])
    tok_src3 = jnp.concatenate([
        tok_src[:4096].reshape(NW, NCHUNK, SUB),
        jnp.broadcast_to(tok_src[4096], (NW, 1, SUB)),
    ], axis=1)
    out = _sc_gather(
        embed_table,
        image_embeds[0],
        tok_src3,
        jnp.asarray(_TOK_DST3),
        jnp.asarray(_IMG_SRC3),
        jnp.asarray(_IMG_DST3),
    )
    return lax.optimization_barrier(out)
